# Initial kernel scaffold; baseline (speedup 1.0000x reference)
import jax
import jax.numpy as jnp
from jax.experimental import pallas as pl


def kernel(user_idx, item_idx, user_emb_gmf, item_emb_gmf, user_emb_mlp, item_emb_mlp, item_bert, W1, b1, W2, b2):
    raise NotImplementedError("write your pallas kernel here")


# speedup vs baseline: 1.8609x; 1.8609x over previous
import functools

import jax
import jax.numpy as jnp
from jax import lax
from jax.experimental import pallas as pl
from jax.experimental.pallas import tpu as pltpu
from jax.experimental.pallas import tpu_sc as plsc

B = 16384
MF = 64
HALF = 64
BD = 768
H1 = 64
CW = MF + HALF

NC = 2
NS = 16
NW = NC * NS
BPW = B // NW
CH = 64
NCH = BPW // CH


def _sc_gather(user_idx, item_idx, utab, itab, bert):
    mesh = plsc.VectorSubcoreMesh(core_axis_name="c", subcore_axis_name="s")
    out_type = [
        jax.ShapeDtypeStruct((B, CW), jnp.float32),
        jax.ShapeDtypeStruct((B, CW), jnp.float32),
        jax.ShapeDtypeStruct((B, BD), jnp.float32),
    ]
    scratch_types = [
        pltpu.VMEM((BPW,), jnp.int32),
        pltpu.VMEM((BPW,), jnp.int32),
        pltpu.VMEM((CH, CW), jnp.float32),
        pltpu.VMEM((CH, CW), jnp.float32),
        pltpu.VMEM((CH, BD), jnp.float32),
        pltpu.SemaphoreType.DMA,
    ]

    @functools.partial(pl.kernel, mesh=mesh, out_type=out_type,
                       scratch_types=scratch_types)
    def k(uidx_h, iidx_h, utab_h, itab_h, bert_h,
          ou_h, oi_h, obert_h,
          uidx_v, iidx_v, bu_v, bi_v, bbert_v, sem):
        wid = lax.axis_index("s") * NC + lax.axis_index("c")
        base = wid * BPW
        pltpu.sync_copy(uidx_h.at[pl.ds(base, BPW)], uidx_v)
        pltpu.sync_copy(iidx_h.at[pl.ds(base, BPW)], iidx_v)

        @pl.loop(0, NCH)
        def _(ci):
            off = ci * CH
            ui = uidx_v.at[pl.ds(off, CH)]
            ii = iidx_v.at[pl.ds(off, CH)]
            c1 = pltpu.async_copy(utab_h.at[ui], bu_v, sem)
            c2 = pltpu.async_copy(itab_h.at[ii], bi_v, sem)
            c3 = pltpu.async_copy(bert_h.at[ii], bbert_v, sem)
            c1.wait(); c2.wait(); c3.wait()
            dst = pl.ds(base + off, CH)
            pltpu.sync_copy(bu_v, ou_h.at[dst])
            pltpu.sync_copy(bi_v, oi_h.at[dst])
            pltpu.sync_copy(bbert_v, obert_h.at[dst])

    return k(user_idx, item_idx, utab, itab, bert)


BT = 512


def _tc_body(u_r, i_r, bt_r, w1u_r, w1i_r, w1b_r, b1_r,
             w2a_r, w2b_r, b2_r, o_r):
    dot = functools.partial(jnp.dot, preferred_element_type=jnp.float32,
                            precision=lax.Precision.HIGHEST)
    u = u_r[...]
    it = i_r[...]
    h = (dot(u[:, MF:], w1u_r[...]) + dot(it[:, MF:], w1i_r[...])
         + dot(bt_r[...], w1b_r[...]) + b1_r[...])
    h = jnp.maximum(h, 0.0)
    g = u[:, :MF] * it[:, :MF]
    o = (jnp.sum(g * w2a_r[...], axis=1, keepdims=True)
         + jnp.sum(h * w2b_r[...], axis=1, keepdims=True) + b2_r[...])
    o_r[...] = o


def _tc_compute(ug, ig, bertg, W1, b1, W2, b2, interpret=False):
    w1u = W1[:HALF]
    w1i = W1[HALF:2 * HALF]
    w1b = W1[2 * HALF:]
    b1r = b1.reshape(1, H1)
    w2a = W2[:MF, 0].reshape(1, MF)
    w2b = W2[MF:, 0].reshape(1, H1)
    b2r = b2.reshape(1, 1)

    grid = (B // BT,)
    row = lambda i: (i, 0)
    fixed = lambda i: (0, 0)
    return pl.pallas_call(
        _tc_body,
        grid=grid,
        in_specs=[
            pl.BlockSpec((BT, CW), row),
            pl.BlockSpec((BT, CW), row),
            pl.BlockSpec((BT, BD), row),
            pl.BlockSpec((HALF, H1), fixed),
            pl.BlockSpec((HALF, H1), fixed),
            pl.BlockSpec((BD, H1), fixed),
            pl.BlockSpec((1, H1), fixed),
            pl.BlockSpec((1, MF), fixed),
            pl.BlockSpec((1, H1), fixed),
            pl.BlockSpec((1, 1), fixed),
        ],
        out_specs=pl.BlockSpec((BT, 1), row),
        out_shape=jax.ShapeDtypeStruct((B, 1), jnp.float32),
        interpret=interpret,
    )(ug, ig, bertg, w1u, w1i, w1b, b1r, w2a, w2b, b2r)


def kernel(user_idx, item_idx, user_emb_gmf, item_emb_gmf, user_emb_mlp,
           item_emb_mlp, item_bert, W1, b1, W2, b2):
    user_idx = user_idx.astype(jnp.int32)
    item_idx = item_idx.astype(jnp.int32)
    utab = jnp.concatenate([user_emb_gmf, user_emb_mlp], axis=1)
    itab = jnp.concatenate([item_emb_gmf, item_emb_mlp], axis=1)
    ug, ig, bertg = _sc_gather(user_idx, item_idx, utab, itab, item_bert)
    return _tc_compute(ug, ig, bertg, W1, b1, W2, b2)
